# baseline (speedup 1.0000x reference)
import jax
import jax.numpy as jnp
from jax.experimental import pallas as pl
from jax.experimental.pallas import tpu as pltpu

D3 = 256
D_MODEL = 768
SEQ = 81


def _pe_add_kernel(rows_ref, cols_ref, boxes_ref,
                   row_tab_ref, col_tab_ref, box_tab_ref,
                   x_ref, out_ref, pe_ref):
    @pl.when(pl.program_id(0) == 0)
    def _build_pe():
        for p in range(SEQ):
            r = rows_ref[p]
            c = cols_ref[p]
            bx = boxes_ref[p]
            pe_ref[pl.ds(p, 1), 0:D3] = row_tab_ref[pl.ds(r, 1), :]
            pe_ref[pl.ds(p, 1), D3:2 * D3] = col_tab_ref[pl.ds(c, 1), :]
            pe_ref[pl.ds(p, 1), 2 * D3:D_MODEL] = box_tab_ref[pl.ds(bx, 1), :]

    cell = pl.program_id(0)
    out_ref[...] = x_ref[...] + pe_ref[pl.ds(cell, 1), :]


HALF = 2048


@jax.jit
def kernel(x, row_table, col_table, box_table, rows, cols, boxes):
    b = x.shape[0]
    x2 = x.transpose(1, 0, 2).reshape(SEQ * b, D_MODEL)
    smem = pl.BlockSpec(memory_space=pltpu.SMEM)
    full = lambda shape: pl.BlockSpec(shape, lambda i, j: (0,) * len(shape))
    out2 = pl.pallas_call(
        _pe_add_kernel,
        grid=(SEQ, 2),
        in_specs=[
            smem, smem, smem,
            full((9, D3)),
            full((9, D3)),
            full((9, D_MODEL - 2 * D3)),
            pl.BlockSpec((HALF, D_MODEL), lambda i, j: (i * 2 + j, 0)),
        ],
        out_specs=pl.BlockSpec((HALF, D_MODEL), lambda i, j: (i * 2 + j, 0)),
        out_shape=jax.ShapeDtypeStruct(x2.shape, x2.dtype),
        scratch_shapes=[pltpu.VMEM((SEQ, D_MODEL), jnp.float32)],
        compiler_params=pltpu.CompilerParams(
            dimension_semantics=("arbitrary", "arbitrary"),
        ),
    )(rows, cols, boxes, row_table, col_table, box_table, x2)
    return out2.reshape(SEQ, b, D_MODEL).transpose(1, 0, 2)


# speedup vs baseline: 1.0039x; 1.0039x over previous
import jax
import jax.numpy as jnp
from jax.experimental import pallas as pl
from jax.experimental.pallas import tpu as pltpu

D3 = 256
D_MODEL = 768
SEQ = 81


def _pe_add_kernel(rows_ref, cols_ref, boxes_ref,
                   row_tab_ref, col_tab_ref, box_tab_ref,
                   x_ref, out_ref, pe_ref):
    @pl.when(pl.program_id(0) == 0)
    def _build_pe():
        for p in range(SEQ):
            r = rows_ref[p]
            c = cols_ref[p]
            bx = boxes_ref[p]
            pe_ref[pl.ds(p, 1), 0:D3] = row_tab_ref[pl.ds(r, 1), :]
            pe_ref[pl.ds(p, 1), D3:2 * D3] = col_tab_ref[pl.ds(c, 1), :]
            pe_ref[pl.ds(p, 1), 2 * D3:D_MODEL] = box_tab_ref[pl.ds(bx, 1), :]

    cell = pl.program_id(0)
    out_ref[...] = x_ref[...] + pe_ref[pl.ds(cell, 1), :]


@jax.jit
def kernel(x, row_table, col_table, box_table, rows, cols, boxes):
    b = x.shape[0]
    x2 = x.transpose(1, 0, 2).reshape(SEQ * b, D_MODEL)
    smem = pl.BlockSpec(memory_space=pltpu.SMEM)
    full = lambda shape: pl.BlockSpec(shape, lambda i: (0,) * len(shape))
    out2 = pl.pallas_call(
        _pe_add_kernel,
        grid=(SEQ,),
        in_specs=[
            smem, smem, smem,
            full((9, D3)),
            full((9, D3)),
            full((9, D_MODEL - 2 * D3)),
            pl.BlockSpec((b, D_MODEL), lambda i: (i, 0)),
        ],
        out_specs=pl.BlockSpec((b, D_MODEL), lambda i: (i, 0)),
        out_shape=jax.ShapeDtypeStruct(x2.shape, x2.dtype),
        scratch_shapes=[pltpu.VMEM((SEQ, D_MODEL), jnp.float32)],
        compiler_params=pltpu.CompilerParams(
            dimension_semantics=("arbitrary",),
        ),
    )(rows, cols, boxes, row_table, col_table, box_table, x2)
    return out2.reshape(SEQ, b, D_MODEL).transpose(1, 0, 2)
